# baseline (speedup 1.0000x reference)
import functools

import jax
import jax.numpy as jnp
from jax import lax
from jax.experimental import pallas as pl
from jax.experimental.pallas import tpu as pltpu
from jax.experimental.pallas import tpu_sc as plsc

N = 257
HEAD = 16
TABLE = 964
TABLE_WORDS = TABLE * HEAD
NC, NS = 2, 16
NW = NC * NS
RPAD = 264
CPAD = 272
GROUPS = CPAD // 16


def _row(idx_v, pbt_v, out_v, r, c):
    ids = idx_v[r, pl.ds(c, 16)]
    for h in range(HEAD):
        out_v[h, r, pl.ds(c, 16)] = plsc.load_gather(pbt_v, [ids + h * TABLE])


def _bias_gather(
    idx_hbm, pbt_hbm, out_hbm, idx_v, idx2_v, pbt_v, out_v, sem_i, sem_i2, sem_t
):
    wid = lax.axis_index("s") * NC + lax.axis_index("c")
    r0 = wid * 8
    cp_i = pltpu.async_copy(idx_hbm.at[pl.ds(r0, 8), :], idx_v, sem_i)
    cp_t = pltpu.async_copy(pbt_hbm, pbt_v, sem_t)
    cp_i2 = pltpu.async_copy(idx_hbm.at[pl.ds(RPAD - 8, 8), :], idx2_v, sem_i2)
    cp_i.wait()
    cp_t.wait()

    for r in range(8):
        @plsc.parallel_loop(0, CPAD, step=16, unroll=4)
        def _(c, r=r):
            _row(idx_v, pbt_v, out_v, r, c)

    pltpu.sync_copy(out_v, out_hbm.at[:, pl.ds(r0, 8), :])
    cp_i2.wait()

    @pl.when(wid == 0)
    def _():
        @plsc.parallel_loop(0, CPAD, step=16, unroll=4)
        def _(c):
            _row(idx2_v, pbt_v, out_v, 0, c)

        pltpu.sync_copy(out_v, out_hbm.at[:, pl.ds(RPAD - 8, 8), :])


@functools.partial(
    pl.kernel,
    out_type=jax.ShapeDtypeStruct((HEAD, RPAD, CPAD), jnp.float32),
    mesh=plsc.VectorSubcoreMesh(
        core_axis_name="c", subcore_axis_name="s", num_cores=NC, num_subcores=NS
    ),
    scratch_types=[
        pltpu.VMEM((8, CPAD), jnp.int32),
        pltpu.VMEM((8, CPAD), jnp.int32),
        pltpu.VMEM((TABLE_WORDS,), jnp.float32),
        pltpu.VMEM((HEAD, 8, CPAD), jnp.float32),
        pltpu.SemaphoreType.DMA,
        pltpu.SemaphoreType.DMA,
        pltpu.SemaphoreType.DMA,
    ],
    compiler_params=pltpu.CompilerParams(needs_layout_passes=False),
)
def _bias_gather_call(
    idx_hbm, pbt_hbm, out_hbm, idx_v, idx2_v, pbt_v, out_v, sem_i, sem_i2, sem_t
):
    _bias_gather(
        idx_hbm, pbt_hbm, out_hbm, idx_v, idx2_v, pbt_v, out_v, sem_i, sem_i2, sem_t
    )


def kernel(pb, idx):
    pbt = jnp.ravel(jnp.transpose(pb))
    idxp = jnp.pad(idx, ((0, RPAD - N), (0, CPAD - N)))
    out = _bias_gather_call(idxp, pbt)
    return out[:, :N, :N]


# speedup vs baseline: 1.1446x; 1.1446x over previous
import functools

import jax
import jax.numpy as jnp
from jax import lax
from jax.experimental import pallas as pl
from jax.experimental.pallas import tpu as pltpu
from jax.experimental.pallas import tpu_sc as plsc

N = 257
HEAD = 16
TABLE = 964
TABLE_WORDS = TABLE * HEAD
NC, NS = 2, 16
NW = NC * NS
RPAD = 264
CPAD = 272
GROUPS = CPAD // 16


def _row(idx_v, pbt_v, out_v, r, c):
    ids = idx_v[r, pl.ds(c, 16)]
    for h in range(HEAD):
        out_v[h, r, pl.ds(c, 16)] = plsc.load_gather(pbt_v, [ids + h * TABLE])


def _bias_gather(
    idx_hbm, pbt_hbm, out_hbm, idx_v, idx2_v, pbt_v, out_v, sem_i, sem_i2, sem_t
):
    wid = lax.axis_index("s") * NC + lax.axis_index("c")
    r0 = wid * 8
    cp_i = pltpu.async_copy(idx_hbm.at[pl.ds(r0, 8), :], idx_v, sem_i)
    cp_t = pltpu.async_copy(pbt_hbm, pbt_v, sem_t)
    cp_i2 = pltpu.async_copy(idx_hbm.at[pl.ds(RPAD - 8, 8), :], idx2_v, sem_i2)
    cp_i.wait()
    cp_t.wait()

    @plsc.parallel_loop(0, 8 * CPAD, step=16, unroll=4)
    def _(off):
        r = off // CPAD
        c = lax.rem(off, CPAD)
        _row(idx_v, pbt_v, out_v, r, c)

    pltpu.sync_copy(out_v, out_hbm.at[:, pl.ds(r0, 8), :])
    cp_i2.wait()

    @pl.when(wid == 0)
    def _():
        @plsc.parallel_loop(0, CPAD, step=16, unroll=4)
        def _(c):
            _row(idx2_v, pbt_v, out_v, 0, c)

        pltpu.sync_copy(out_v, out_hbm.at[:, pl.ds(RPAD - 8, 8), :])


@functools.partial(
    pl.kernel,
    out_type=jax.ShapeDtypeStruct((HEAD, RPAD, CPAD), jnp.float32),
    mesh=plsc.VectorSubcoreMesh(
        core_axis_name="c", subcore_axis_name="s", num_cores=NC, num_subcores=NS
    ),
    scratch_types=[
        pltpu.VMEM((8, CPAD), jnp.int32),
        pltpu.VMEM((8, CPAD), jnp.int32),
        pltpu.VMEM((TABLE_WORDS,), jnp.float32),
        pltpu.VMEM((HEAD, 8, CPAD), jnp.float32),
        pltpu.SemaphoreType.DMA,
        pltpu.SemaphoreType.DMA,
        pltpu.SemaphoreType.DMA,
    ],
    compiler_params=pltpu.CompilerParams(needs_layout_passes=False),
)
def _bias_gather_call(
    idx_hbm, pbt_hbm, out_hbm, idx_v, idx2_v, pbt_v, out_v, sem_i, sem_i2, sem_t
):
    _bias_gather(
        idx_hbm, pbt_hbm, out_hbm, idx_v, idx2_v, pbt_v, out_v, sem_i, sem_i2, sem_t
    )


def kernel(pb, idx):
    pbt = jnp.ravel(jnp.transpose(pb))
    idxp = jnp.pad(idx, ((0, RPAD - N), (0, CPAD - N)))
    out = _bias_gather_call(idxp, pbt)
    return out[:, :N, :N]


# speedup vs baseline: 1.1462x; 1.0014x over previous
import functools

import jax
import jax.numpy as jnp
from jax import lax
from jax.experimental import pallas as pl
from jax.experimental.pallas import tpu as pltpu
from jax.experimental.pallas import tpu_sc as plsc

N = 257
HEAD = 16
TABLE = 964
TABLE_WORDS = TABLE * HEAD
NC, NS = 2, 16
RPAD = 264
CPAD = 272


def _bias_gather(idx_hbm, pbt_hbm, out_hbm, idx_v, pbt_v, out_v, sem_i, sem_i2, sem_t):
    wid = lax.axis_index("s") * NC + lax.axis_index("c")
    r0 = wid * 8
    cp_i = pltpu.async_copy(
        idx_hbm.at[pl.ds(r0, 8), :], idx_v.at[pl.ds(0, 8), :], sem_i
    )
    cp_i2 = pltpu.async_copy(
        idx_hbm.at[pl.ds(RPAD - 8, 8), :], idx_v.at[pl.ds(8, 8), :], sem_i2
    )
    cp_t = pltpu.async_copy(pbt_hbm, pbt_v, sem_t)
    cp_i.wait()
    cp_i2.wait()
    cp_t.wait()

    @plsc.parallel_loop(0, 9 * CPAD, step=16, unroll=4)
    def _(off):
        r = off // CPAD
        c = lax.rem(off, CPAD)
        ids = idx_v[r, pl.ds(c, 16)]
        for h in range(HEAD):
            out_v[h, r, pl.ds(c, 16)] = plsc.load_gather(pbt_v, [ids + h * TABLE])

    pltpu.sync_copy(out_v.at[:, pl.ds(0, 8), :], out_hbm.at[:, pl.ds(r0, 8), :])

    @pl.when(wid == 0)
    def _():
        pltpu.sync_copy(
            out_v.at[:, pl.ds(8, 8), :], out_hbm.at[:, pl.ds(RPAD - 8, 8), :]
        )


@functools.partial(
    pl.kernel,
    out_type=jax.ShapeDtypeStruct((HEAD, RPAD, CPAD), jnp.float32),
    mesh=plsc.VectorSubcoreMesh(
        core_axis_name="c", subcore_axis_name="s", num_cores=NC, num_subcores=NS
    ),
    scratch_types=[
        pltpu.VMEM((16, CPAD), jnp.int32),
        pltpu.VMEM((TABLE_WORDS,), jnp.float32),
        pltpu.VMEM((HEAD, 16, CPAD), jnp.float32),
        pltpu.SemaphoreType.DMA,
        pltpu.SemaphoreType.DMA,
        pltpu.SemaphoreType.DMA,
    ],
    compiler_params=pltpu.CompilerParams(needs_layout_passes=False),
)
def _bias_gather_call(idx_hbm, pbt_hbm, out_hbm, idx_v, pbt_v, out_v, sem_i, sem_i2, sem_t):
    _bias_gather(idx_hbm, pbt_hbm, out_hbm, idx_v, pbt_v, out_v, sem_i, sem_i2, sem_t)


def kernel(pb, idx):
    pbt = jnp.ravel(jnp.transpose(pb))
    idxp = jnp.pad(idx, ((0, RPAD - N), (0, CPAD - N)))
    out = _bias_gather_call(idxp, pbt)
    return out[:, :N, :N]
